# baseline (speedup 1.0000x reference)
import functools

import jax
import jax.numpy as jnp
from jax import lax
from jax.experimental import pallas as pl
from jax.experimental.pallas import tpu as pltpu
from jax.experimental.pallas import tpu_sc as plsc

VOCAB = 1_000_000
D = 64
BATCH = 4096
SEQ = 200

NC = 2
NS = 16
NW = NC * NS
B_PER_W = BATCH // NW
NBUF = 5
DEPTH = 3

_mesh = plsc.VectorSubcoreMesh(
    core_axis_name="c", subcore_axis_name="s", num_cores=NC, num_subcores=NS
)


@functools.partial(
    pl.kernel,
    out_type=jax.ShapeDtypeStruct((SEQ, 8, NW, 8, 128), jnp.float32),
    mesh=_mesh,
    scratch_types=[
        pltpu.VMEM((SEQ, B_PER_W), jnp.int32),
        pltpu.VMEM((NBUF, B_PER_W, D), jnp.float32),
        pltpu.VMEM((NBUF, 8, 8, 128), jnp.float32),
    ] + [pltpu.SemaphoreType.DMA] * (2 * NBUF),
    compiler_params=pltpu.CompilerParams(
        use_tc_tiling_on_sc=False, needs_layout_passes=False
    ),
)
def _sc_gather(table_hbm, idx_hbm, out_hbm, idx_v, rows_v, ct_v, *sems):
    gat_sems = sems[:NBUF]
    st_sems = sems[NBUF:]
    wid = lax.axis_index("s") * NC + lax.axis_index("c")

    pltpu.sync_copy(idx_hbm.at[:, pl.ds(wid * B_PER_W, B_PER_W)], idx_v)

    lane = lax.iota(jnp.int32, 16)

    def issue_gather(i, b):
        pltpu.async_copy(
            table_hbm.at[idx_v.at[i]], rows_v.at[b], gat_sems[b]
        )

    def wait_gather(b):
        pltpu.make_async_copy(
            table_hbm.at[pl.ds(0, B_PER_W)], rows_v.at[b], gat_sems[b]
        ).wait()

    def transform(b):
        rv = rows_v.at[b]

        def tc_body(tc, carry):
            for cs in range(8):
                col = jnp.full((16,), 8 * tc + cs, jnp.int32)
                for k in range(8):
                    vals = plsc.load_gather(rv, [lane + 16 * k, col])
                    ct_v[b, tc, cs, pl.ds(16 * k, 16)] = vals
            return carry

        lax.fori_loop(0, 8, tc_body, 0)

    def issue_store(i, b):
        pltpu.async_copy(
            ct_v.at[b], out_hbm.at[i, :, wid], st_sems[b]
        )

    def wait_store(b):
        pltpu.make_async_copy(
            ct_v.at[b], out_hbm.at[0, :, 0], st_sems[b]
        ).wait()

    for i in range(DEPTH):
        issue_gather(i, i)

    def slot(i, b, b4, first, last):
        wait_gather(b)
        transform(b)
        issue_store(i, b)
        if not last:
            if not first:
                wait_store(b4)
            issue_gather(i + DEPTH, b4)

    for i in range(2):
        slot(i, i % NBUF, (i + DEPTH) % NBUF, True, False)

    def body(k, carry):
        i0 = 2 + NBUF * k
        for j in range(NBUF):
            slot(i0 + j, (2 + j) % NBUF, (2 + j + DEPTH) % NBUF, False, False)
        return carry

    nb = (SEQ - DEPTH - 2) // NBUF
    lax.fori_loop(0, nb, body, 0)
    for i in range(2 + NBUF * nb, SEQ - DEPTH):
        slot(i, i % NBUF, (i + DEPTH) % NBUF, False, False)

    for i in range(SEQ - DEPTH, SEQ):
        slot(i, i % NBUF, 0, False, True)
    for b in range(NBUF):
        wait_store(b)


def kernel(x, table):
    out5 = _sc_gather(table, x.T)
    return out5.transpose(2, 4, 0, 1, 3).reshape(BATCH, SEQ, D)


# speedup vs baseline: 1.5792x; 1.5792x over previous
import functools

import jax
import jax.numpy as jnp
from jax import lax
from jax.experimental import pallas as pl
from jax.experimental.pallas import tpu as pltpu
from jax.experimental.pallas import tpu_sc as plsc

VOCAB = 1_000_000
D = 64
BATCH = 4096
SEQ = 200

NC = 2
NS = 16
NW = NC * NS
B_PER_W = BATCH // NW
NBUF = 5
DEPTH = 3

_mesh = plsc.VectorSubcoreMesh(
    core_axis_name="c", subcore_axis_name="s", num_cores=NC, num_subcores=NS
)


@functools.partial(
    pl.kernel,
    out_type=jax.ShapeDtypeStruct((SEQ, BATCH, D), jnp.float32),
    mesh=_mesh,
    scratch_types=[
        pltpu.VMEM((SEQ, B_PER_W), jnp.int32),
        pltpu.VMEM((NBUF, B_PER_W, D), jnp.float32),
    ] + [pltpu.SemaphoreType.DMA] * (2 * NBUF),
    compiler_params=pltpu.CompilerParams(
        use_tc_tiling_on_sc=False, needs_layout_passes=False
    ),
)
def _sc_gather(table_hbm, idx_hbm, out_hbm, idx_v, rows_v, *sems):
    gat_sems = sems[:NBUF]
    st_sems = sems[NBUF:]
    wid = lax.axis_index("s") * NC + lax.axis_index("c")

    pltpu.sync_copy(idx_hbm.at[:, pl.ds(wid * B_PER_W, B_PER_W)], idx_v)

    lane = lax.iota(jnp.int32, 16)

    def issue_gather(i, b):
        pltpu.async_copy(
            table_hbm.at[idx_v.at[i]], rows_v.at[b], gat_sems[b]
        )

    def wait_gather(b):
        pltpu.make_async_copy(
            table_hbm.at[pl.ds(0, B_PER_W)], rows_v.at[b], gat_sems[b]
        ).wait()

    def transform(b):
        pass

    def issue_store(i, b):
        pltpu.async_copy(
            rows_v.at[b], out_hbm.at[i, pl.ds(wid * B_PER_W, B_PER_W)],
            st_sems[b],
        )

    def wait_store(b):
        pltpu.make_async_copy(
            rows_v.at[b], out_hbm.at[0, pl.ds(0, B_PER_W)], st_sems[b]
        ).wait()

    for i in range(DEPTH):
        issue_gather(i, i)

    def slot(i, b, b4, first, last):
        wait_gather(b)
        transform(b)
        issue_store(i, b)
        if not last:
            if not first:
                wait_store(b4)
            issue_gather(i + DEPTH, b4)

    for i in range(2):
        slot(i, i % NBUF, (i + DEPTH) % NBUF, True, False)

    def body(k, carry):
        i0 = 2 + NBUF * k
        for j in range(NBUF):
            slot(i0 + j, (2 + j) % NBUF, (2 + j + DEPTH) % NBUF, False, False)
        return carry

    nb = (SEQ - DEPTH - 2) // NBUF
    lax.fori_loop(0, nb, body, 0)
    for i in range(2 + NBUF * nb, SEQ - DEPTH):
        slot(i, i % NBUF, (i + DEPTH) % NBUF, False, False)

    for i in range(SEQ - DEPTH, SEQ):
        slot(i, i % NBUF, 0, False, True)
    for b in range(NBUF):
        wait_store(b)


def kernel(x, table):
    out_sb = _sc_gather(table, x.T)
    return out_sb.transpose(1, 0, 2)
